# baseline (speedup 1.0000x reference)
import functools
import math

import jax
import jax.numpy as jnp
from jax import lax
from jax.experimental import pallas as pl
from jax.experimental.pallas import tpu as pltpu
from jax.experimental.pallas import tpu_sc as plsc

D_MODEL_K = 4096
MAX_NORM_K = math.sqrt(D_MODEL_K)
CHUNK = 1024
NBUF = 3
_LANES = 16
SC_CHUNK = 8


def _rsqrt_scalar(s):
    i = lax.bitcast_convert_type(s, jnp.int32)
    i = jnp.int32(0x5F3759DF) - lax.shift_right_arithmetic(i, 1)
    y = lax.bitcast_convert_type(i, jnp.float32)
    for _ in range(4):
        y = y * (1.5 - 0.5 * s * y * y)
    return y


def _sc_add_kernel(ts_hbm, tbl_hbm, x_hbm, o_hbm, idx_v, rows_v, emb1, buf,
                   gsem, isem, osem, *, row_base, rows_per_w, n_batch,
                   d_model):
    nc = plsc.get_sparse_core_info().num_cores
    wid = lax.axis_index("s") * nc + lax.axis_index("c")
    nvec = d_model // _LANES
    b = n_batch - 1

    pltpu.sync_copy(ts_hbm, idx_v)
    pltpu.async_copy(tbl_hbm.at[idx_v], rows_v, gsem).wait()

    def sumsq(j, acc):
        v = rows_v[b, pl.ds(j * _LANES, _LANES)]
        return acc + v * v

    acc = lax.fori_loop(0, nvec, sumsq, jnp.zeros((_LANES,), jnp.float32),
                        unroll=8)
    s = acc[0]
    for k in range(1, _LANES):
        s = s + acc[k]
    scale = jnp.where(s > jnp.float32(MAX_NORM_K * MAX_NORM_K),
                      MAX_NORM_K * _rsqrt_scalar(s), jnp.float32(1.0))

    def rescale(j, c):
        sl = pl.ds(j * _LANES, _LANES)
        emb1[sl] = rows_v[b, sl] * scale
        return c

    lax.fori_loop(0, nvec, rescale, 0, unroll=8)

    my0 = row_base + wid * rows_per_w
    n_chunks = rows_per_w // SC_CHUNK

    def copy_in(c, slot):
        return pltpu.make_async_copy(
            x_hbm.at[pl.ds(my0 + c * SC_CHUNK, SC_CHUNK), :],
            buf.at[slot], isem.at[slot])

    def copy_out(c, slot):
        return pltpu.make_async_copy(
            buf.at[slot],
            o_hbm.at[pl.ds(wid * rows_per_w + c * SC_CHUNK, SC_CHUNK), :],
            osem.at[slot])

    copy_in(0, 0).start()
    copy_in(1, 1).start()
    for c in range(n_chunks):
        slot = c % 2
        copy_in(c, slot).wait()

        for r in range(SC_CHUNK):
            def add_row(j, c2, _r=r, _slot=slot):
                sl = pl.ds(j * _LANES, _LANES)
                buf[_slot, _r, sl] = buf[_slot, _r, sl] + emb1[sl]
                return c2

            lax.fori_loop(0, nvec, add_row, 0, unroll=8)
        copy_out(c, slot).start()
        if c + 2 < n_chunks:
            copy_out(c, slot).wait()
            copy_in(c + 2, slot).start()
    copy_out(n_chunks - 2, n_chunks % 2).wait()
    copy_out(n_chunks - 1, (n_chunks - 1) % 2).wait()


def _sc_add_last_batch(timesteps, table, x2, row_base, rows):
    B = timesteps.shape[0]
    D = table.shape[1]
    info = plsc.get_sparse_core_info()
    n_workers = info.num_cores * info.num_subcores
    rows_per_w = rows // n_workers
    mesh = plsc.VectorSubcoreMesh(core_axis_name="c", subcore_axis_name="s")
    return pl.kernel(
        functools.partial(_sc_add_kernel, row_base=row_base,
                          rows_per_w=rows_per_w, n_batch=B, d_model=D),
        out_type=jax.ShapeDtypeStruct((rows, D), x2.dtype),
        mesh=mesh,
        scratch_types=[
            pltpu.VMEM((B,), jnp.int32),
            pltpu.VMEM((B, D), table.dtype),
            pltpu.VMEM((D,), table.dtype),
            pltpu.VMEM((2, SC_CHUNK, D), x2.dtype),
            pltpu.SemaphoreType.DMA,
            pltpu.SemaphoreType.DMA((2,)),
            pltpu.SemaphoreType.DMA((2,)),
        ],
    )(timesteps, table, x2)


def _tc_pipeline(ts_ref, x_hbm, tbl_hbm, o_hbm, buf, emb_ref, in_sems,
                 out_sems, row_sem, *, n_chunks, chunks_per_b, n_batch):
    for b in range(n_batch):
        pltpu.make_async_copy(
            tbl_hbm.at[pl.ds(ts_ref[b], 1), :], emb_ref.at[pl.ds(b, 1), :],
            row_sem,
        ).start()

    def copy_in(c, slot):
        return pltpu.make_async_copy(
            x_hbm.at[pl.ds(c * CHUNK, CHUNK), :], buf.at[slot],
            in_sems.at[slot])

    def copy_out(c, slot):
        return pltpu.make_async_copy(
            buf.at[slot], o_hbm.at[pl.ds(c * CHUNK, CHUNK), :],
            out_sems.at[slot])

    for s in range(min(NBUF, n_chunks)):
        copy_in(s, s).start()

    for b in range(n_batch):
        pltpu.make_async_copy(
            tbl_hbm.at[pl.ds(ts_ref[b], 1), :], emb_ref.at[pl.ds(b, 1), :],
            row_sem,
        ).wait()
    rows = emb_ref[...]
    norms = jnp.sqrt(jnp.sum(rows * rows, axis=-1, keepdims=True))
    emb_ref[...] = rows * jnp.where(norms > MAX_NORM_K,
                                    MAX_NORM_K / (norms + 1e-7), 1.0)

    for c in range(n_chunks):
        slot = c % NBUF
        b = c // chunks_per_b
        copy_in(c, slot).wait()
        buf[slot] += emb_ref[pl.ds(b, 1), :]
        copy_out(c, slot).start()
        nxt = c + NBUF
        if nxt < n_chunks:
            copy_out(c, slot).wait()
            copy_in(nxt, slot).start()

    for c in range(max(0, n_chunks - NBUF), n_chunks):
        copy_out(c, c % NBUF).wait()


def _tc_add_batches(timesteps, table, x2, tc_rows):
    B = timesteps.shape[0]
    D = table.shape[1]
    n_chunks = tc_rows // CHUNK
    chunks_per_b = 2048 // CHUNK
    body = functools.partial(_tc_pipeline, n_chunks=n_chunks,
                             chunks_per_b=chunks_per_b, n_batch=B)
    return pl.pallas_call(
        body,
        grid_spec=pltpu.PrefetchScalarGridSpec(
            num_scalar_prefetch=1,
            grid=(1,),
            in_specs=[
                pl.BlockSpec(memory_space=pl.ANY),
                pl.BlockSpec(memory_space=pl.ANY),
            ],
            out_specs=pl.BlockSpec(memory_space=pl.ANY),
            scratch_shapes=[
                pltpu.VMEM((NBUF, CHUNK, D), x2.dtype),
                pltpu.VMEM((B, D), x2.dtype),
                pltpu.SemaphoreType.DMA((NBUF,)),
                pltpu.SemaphoreType.DMA((NBUF,)),
                pltpu.SemaphoreType.DMA,
            ],
        ),
        out_shape=jax.ShapeDtypeStruct((tc_rows, D), x2.dtype),
    )(timesteps, x2, table)


def kernel(x, timesteps, table):
    B, S, D = x.shape
    x2 = x.reshape(B * S, D)
    sc_rows = S
    tc_rows = B * S - sc_rows
    out_tc = _tc_add_batches(timesteps, table, x2, tc_rows)
    out_sc = _sc_add_last_batch(timesteps, table, x2, tc_rows, sc_rows)
    return jnp.concatenate([out_tc, out_sc], axis=0).reshape(B, S, D)


# speedup vs baseline: 2.7218x; 2.7218x over previous
import functools
import math

import jax
import jax.numpy as jnp
from jax.experimental import pallas as pl
from jax.experimental.pallas import tpu as pltpu

D_MODEL_K = 4096
MAX_NORM_K = math.sqrt(D_MODEL_K)
CHUNK = 1024
NBUF = 3
NSPLIT = 1


def _pipeline_kernel(ts_ref, x_hbm, tbl_hbm, o_hbm, buf, emb_ref,
                     in_sems, out_sems, row_sem, *, n_chunks, chunks_per_b,
                     n_batch):
    for b in range(n_batch):
        pltpu.make_async_copy(
            tbl_hbm.at[pl.ds(ts_ref[b], 1), :], emb_ref.at[pl.ds(b, 1), :],
            row_sem,
        ).start()

    sub = CHUNK // NSPLIT

    def copies_in(c, slot):
        return [
            pltpu.make_async_copy(
                x_hbm.at[pl.ds(c * CHUNK + k * sub, sub), :],
                buf.at[slot, pl.ds(k * sub, sub), :],
                in_sems.at[slot],
            )
            for k in range(NSPLIT)
        ]

    def copies_out(c, slot):
        return [
            pltpu.make_async_copy(
                buf.at[slot, pl.ds(k * sub, sub), :],
                o_hbm.at[pl.ds(c * CHUNK + k * sub, sub), :],
                out_sems.at[slot],
            )
            for k in range(NSPLIT)
        ]

    def start(cps):
        for cp in cps:
            cp.start()

    def wait(cps):
        for cp in cps:
            cp.wait()

    for s in range(min(NBUF, n_chunks)):
        start(copies_in(s, s))

    for b in range(n_batch):
        pltpu.make_async_copy(
            tbl_hbm.at[pl.ds(ts_ref[b], 1), :], emb_ref.at[pl.ds(b, 1), :],
            row_sem,
        ).wait()
    rows = emb_ref[...]
    norms = jnp.sqrt(jnp.sum(rows * rows, axis=-1, keepdims=True))
    emb_ref[...] = rows * jnp.where(norms > MAX_NORM_K,
                                    MAX_NORM_K / (norms + 1e-7), 1.0)

    for c in range(n_chunks):
        slot = c % NBUF
        b = c // chunks_per_b
        wait(copies_in(c, slot))
        buf[slot] += emb_ref[pl.ds(b, 1), :]
        start(copies_out(c, slot))
        nxt = c + NBUF
        if nxt < n_chunks:
            wait(copies_out(c, slot))
            start(copies_in(nxt, slot))

    for c in range(max(0, n_chunks - NBUF), n_chunks):
        wait(copies_out(c, c % NBUF))


def kernel(x, timesteps, table):
    B, S, D = x.shape
    x2 = x.reshape(B * S, D)
    n_chunks = (B * S) // CHUNK
    chunks_per_b = S // CHUNK
    body = functools.partial(_pipeline_kernel, n_chunks=n_chunks,
                             chunks_per_b=chunks_per_b, n_batch=B)
    out = pl.pallas_call(
        body,
        grid_spec=pltpu.PrefetchScalarGridSpec(
            num_scalar_prefetch=1,
            grid=(1,),
            in_specs=[
                pl.BlockSpec(memory_space=pl.ANY),
                pl.BlockSpec(memory_space=pl.ANY),
            ],
            out_specs=pl.BlockSpec(memory_space=pl.ANY),
            scratch_shapes=[
                pltpu.VMEM((NBUF, CHUNK, D), x.dtype),
                pltpu.VMEM((B, D), x.dtype),
                pltpu.SemaphoreType.DMA((NBUF,)),
                pltpu.SemaphoreType.DMA((NBUF,)),
                pltpu.SemaphoreType.DMA,
            ],
        ),
        out_shape=jax.ShapeDtypeStruct(x2.shape, x.dtype),
        compiler_params=pltpu.CompilerParams(
            vmem_limit_bytes=128 * 1024 * 1024,
        ),
    )(timesteps, x2, table)
    return out.reshape(B, S, D)
